# Initial kernel scaffold; baseline (speedup 1.0000x reference)
import jax
import jax.numpy as jnp
from jax.experimental import pallas as pl


def kernel(hidden_states, router_weight, w_gate_up, w_down):
    raise NotImplementedError("write your pallas kernel here")


# speedup vs baseline: 1.7561x; 1.7561x over previous
import jax
import jax.numpy as jnp
from jax import lax
from jax.experimental import pallas as pl

T = 1024
D = 1024
E = 8
FF = 512


def _combine_col(x, wr, e):
    logits = lax.dot_general(x, wr, (((1,), (1,)), ((), ())),
                             preferred_element_type=jnp.float32)
    probs = jax.nn.softmax(logits, axis=-1)
    col = lax.broadcasted_iota(jnp.int32, probs.shape, 1)
    v1 = jnp.max(probs, axis=-1, keepdims=True)
    i1 = jnp.min(jnp.where(probs == v1, col, E), axis=-1, keepdims=True)
    masked = jnp.where(col == i1, -jnp.inf, probs)
    v2 = jnp.max(masked, axis=-1, keepdims=True)
    i2 = jnp.min(jnp.where(masked == v2, col, E), axis=-1, keepdims=True)
    s = v1 + v2
    w1 = v1 / s
    w2 = v2 / s
    return jnp.where(i1 == e, w1, 0.0) + jnp.where(i2 == e, w2, 0.0)


def _moe_body(x_ref, wr_ref, wgu_ref, wd_ref, out_ref):
    e = pl.program_id(0)
    x = x_ref[...]
    c_e = _combine_col(x, wr_ref[...], e)
    wgu = wgu_ref[0]
    gu = lax.dot_general(x, wgu, (((1,), (1,)), ((), ())),
                         preferred_element_type=jnp.float32)
    gate = gu[:, :FF]
    up = gu[:, FF:]
    act = gate * jax.nn.sigmoid(gate) * up
    wd = wd_ref[0]
    y = lax.dot_general(act, wd, (((1,), (1,)), ((), ())),
                        preferred_element_type=jnp.float32)
    contrib = c_e * y

    @pl.when(e == 0)
    def _():
        out_ref[...] = contrib

    @pl.when(e != 0)
    def _():
        out_ref[...] = out_ref[...] + contrib


def kernel(hidden_states, router_weight, w_gate_up, w_down):
    return pl.pallas_call(
        _moe_body,
        grid=(E,),
        in_specs=[
            pl.BlockSpec((T, D), lambda e: (0, 0)),
            pl.BlockSpec((E, D), lambda e: (0, 0)),
            pl.BlockSpec((1, 2 * FF, D), lambda e: (e, 0, 0)),
            pl.BlockSpec((1, D, FF), lambda e: (e, 0, 0)),
        ],
        out_specs=pl.BlockSpec((T, D), lambda e: (0, 0)),
        out_shape=jax.ShapeDtypeStruct((T, D), jnp.float32),
    )(hidden_states, router_weight, w_gate_up, w_down)
